# baseline (speedup 1.0000x reference)
import functools

import jax
import jax.numpy as jnp
from jax import lax
from jax.experimental import pallas as pl
from jax.experimental.pallas import tpu as pltpu
from jax.experimental.pallas import tpu_sc as plsc

N = 10000
D = 128
DP = D // 2
E = 320000
REG_PARAM = 0.01

NC, NS = 2, 16
EHALF = E // NC
CE = 4096
NCH = 41 * CE
EPAD = EHALF + NCH
NCHUNK2 = EHALF // CE + 1
RPE = EHALF // 256
DUMMY = 2 * RPE
SROWS = 1280


def _sc_scores_body(ubt_hbm, ibt_hbm, wcp_hbm, wbp_hbm,
                    srcc_hbm, dstc_hbm, srcb_hbm, dstb_hbm,
                    outc_hbm, outb_hbm,
                    ubslab, ibslab, wcbuf, wbbuf,
                    srcA, dstA, srcB, dstB, partA, partB, rowidx,
                    shared, semIA, semIB, semPA, semPB):
    cid = lax.axis_index("c")
    fg = lax.axis_index("s")
    lane = lax.iota(jnp.int32, 16)
    zf = jnp.zeros(16, jnp.float32)
    zbf = jnp.zeros(32, jnp.bfloat16)
    base = cid * EHALF

    pltpu.sync_copy(ubt_hbm.at[fg], ubslab)
    pltpu.sync_copy(ibt_hbm.at[fg], ibslab)
    pltpu.sync_copy(wcp_hbm, wcbuf)
    pltpu.sync_copy(wbp_hbm, wbbuf)

    for e in range(2):
        for ch in range(NCHUNK2):
            v = ch * 16 + lane
            rowidx[e * NCHUNK2 + ch, :] = jnp.where(
                v < RPE, v + e * RPE, DUMMY + lane)

    for r in range(16):
        for j in range(16):
            partA[r, pl.ds(j * 16, 16)] = zf
    for k in range(5):
        pltpu.sync_copy(partA, shared.at[pl.ds(fg * 80 + k * 16, 16)])
    plsc.subcore_barrier()

    cvecs = [jnp.full((16,), col, jnp.int32) for col in range(4)]

    def run_etype(e, src_hbm, dst_hbm, wbuf):
        wrow = wbuf[fg, pl.ds(0, 16)]
        wcols = [plsc.bitcast(jnp.broadcast_to(wrow[col], (16,)),
                              jnp.bfloat16) for col in range(4)]

        def fetch(ch, sbuf, dbuf, sem):
            off = base + ch * CE
            pltpu.async_copy(src_hbm.at[pl.ds(off, CE)], sbuf, sem)
            pltpu.async_copy(dst_hbm.at[pl.ds(off, CE)], dbuf, sem)

        def dfetch(sbuf, dbuf, sem):
            pltpu.make_async_copy(src_hbm.at[pl.ds(0, CE)], sbuf, sem).wait()
            pltpu.make_async_copy(dst_hbm.at[pl.ds(0, CE)], dbuf, sem).wait()

        def compute(sbuf, dbuf, part):
            def grp(g, carry):
                src16 = sbuf[pl.ds(g * 16, 16)]
                dst16 = dbuf[pl.ds(g * 16, 16)]
                acc = zbf
                for col in range(4):
                    u = plsc.load_gather(ubslab, [cvecs[col], src16])
                    iv = plsc.load_gather(ibslab, [cvecs[col], dst16])
                    acc = acc + ((plsc.bitcast(u, jnp.bfloat16) * wcols[col])
                                 * plsc.bitcast(iv, jnp.bfloat16))
                hi, lo = plsc.unpack(acc, format=plsc.PackFormat.INTERLEAVED)
                part[g >> 4, pl.ds((g & 15) * 16, 16)] = hi + lo
                return carry

            lax.fori_loop(0, CE // 16, grp, 0, unroll=4)

        def addout(ch, part, sem):
            pltpu.async_copy(part, shared.at[rowidx.at[e * NCHUNK2 + ch]],
                             sem, add=True)

        def dadd(part, sem):
            pltpu.make_async_copy(part, shared.at[pl.ds(0, 16)], sem).wait()

        fetch(0, srcA, dstA, semIA)

        def ch_pair(t, carry):
            ch = 2 * t
            fetch(ch + 1, srcB, dstB, semIB)
            dfetch(srcA, dstA, semIA)

            @pl.when(t > 0)
            def _():
                dadd(partA, semPA)

            compute(srcA, dstA, partA)
            addout(ch, partA, semPA)
            fetch(ch + 2, srcA, dstA, semIA)
            dfetch(srcB, dstB, semIB)

            @pl.when(t > 0)
            def _():
                dadd(partB, semPB)

            compute(srcB, dstB, partB)
            addout(ch + 1, partB, semPB)
            return carry

        lax.fori_loop(0, NCHUNK2 // 2, ch_pair, 0)
        dfetch(srcA, dstA, semIA)
        dadd(partA, semPA)
        dadd(partB, semPB)

    run_etype(0, srcc_hbm, dstc_hbm, wcbuf)
    run_etype(1, srcb_hbm, dstb_hbm, wbbuf)

    plsc.subcore_barrier()

    @pl.when(fg < 15)
    def _():
        pltpu.sync_copy(shared.at[pl.ds(fg * 40, 40)],
                        outc_hbm.at[cid].at[pl.ds(fg * 40, 40)])
        pltpu.sync_copy(shared.at[pl.ds(RPE + fg * 40, 40)],
                        outb_hbm.at[cid].at[pl.ds(fg * 40, 40)])

    @pl.when(fg == 15)
    def _():
        pltpu.sync_copy(shared.at[pl.ds(600, 25)],
                        outc_hbm.at[cid].at[pl.ds(600, 25)])
        pltpu.sync_copy(shared.at[pl.ds(RPE + 600, 25)],
                        outb_hbm.at[cid].at[pl.ds(600, 25)])


_sc_scores = pl.kernel(
    _sc_scores_body,
    out_type=(jax.ShapeDtypeStruct((NC, RPE, 256), jnp.float32),
              jax.ShapeDtypeStruct((NC, RPE, 256), jnp.float32)),
    mesh=plsc.VectorSubcoreMesh(core_axis_name="c", subcore_axis_name="s",
                                num_cores=NC, num_subcores=NS),
    scratch_types=[
        pltpu.VMEM((4, N), jnp.int32),
        pltpu.VMEM((4, N), jnp.int32),
        pltpu.VMEM((16, 16), jnp.int32),
        pltpu.VMEM((16, 16), jnp.int32),
        pltpu.VMEM((CE,), jnp.int32),
        pltpu.VMEM((CE,), jnp.int32),
        pltpu.VMEM((CE,), jnp.int32),
        pltpu.VMEM((CE,), jnp.int32),
        pltpu.VMEM((16, 256), jnp.float32),
        pltpu.VMEM((16, 256), jnp.float32),
        pltpu.VMEM((2 * NCHUNK2, 16), jnp.int32),
        pltpu.VMEM_SHARED((SROWS, 256), jnp.float32),
        pltpu.SemaphoreType.DMA,
        pltpu.SemaphoreType.DMA,
        pltpu.SemaphoreType.DMA,
        pltpu.SemaphoreType.DMA,
    ],
    compiler_params=pltpu.CompilerParams(needs_layout_passes=False,
                                         use_tc_tiling_on_sc=False),
)


def _tc_loss_body(sc_ref, sb_ref, lc_ref, lb_ref, u_ref, i_ref, wc_ref, wb_ref,
                  out_ref):
    def bce_sum(s, y):
        return jnp.sum(jnp.maximum(s, 0.0) - s * y
                       + jnp.log1p(jnp.exp(-jnp.abs(s))))

    predict = (bce_sum(sc_ref[...], lc_ref[...])
               + bce_sum(sb_ref[...], lb_ref[...])) / E
    reg = (jnp.mean(u_ref[...] ** 2) + jnp.mean(i_ref[...] ** 2)
           + jnp.mean(wc_ref[...] ** 2) + jnp.mean(wb_ref[...] ** 2))
    out_ref[...] = jnp.full((1, 1), predict + REG_PARAM * reg, jnp.float32)


_tc_loss = pl.pallas_call(
    _tc_loss_body,
    out_shape=jax.ShapeDtypeStruct((1, 1), jnp.float32),
)


def _pack_t(x_f32):
    p = jax.lax.bitcast_convert_type(
        x_f32.astype(jnp.bfloat16).reshape(N, DP, 2), jnp.int32)
    return p.T.reshape(NS, 4, N)


def _pack_w(w):
    p = jax.lax.bitcast_convert_type(
        w.astype(jnp.bfloat16).reshape(DP, 2), jnp.int32).reshape(NS, 4)
    return jnp.pad(p, ((0, 0), (0, 12)))


def _pad_edges(col):
    return jnp.concatenate([col, jnp.zeros(EPAD - E, jnp.int32)])


def kernel(embed_user, embed_item, edges_click, edges_buy, labels_click,
           labels_buy, w_click, w_buy):
    srcc = _pad_edges(edges_click[:, 0])
    dstc = _pad_edges(edges_click[:, 1])
    srcb = _pad_edges(edges_buy[:, 0])
    dstb = _pad_edges(edges_buy[:, 1])
    scores_c, scores_b = _sc_scores(
        _pack_t(embed_user), _pack_t(embed_item),
        _pack_w(w_click), _pack_w(w_buy),
        srcc, dstc, srcb, dstb)
    out = _tc_loss(scores_c.reshape(E // D, D), scores_b.reshape(E // D, D),
                   labels_click.reshape(E // D, D), labels_buy.reshape(E // D, D),
                   embed_user, embed_item,
                   w_click.reshape(1, D), w_buy.reshape(1, D))
    return out[0, 0]


# speedup vs baseline: 1.9133x; 1.9133x over previous
import functools

import jax
import jax.numpy as jnp
from jax import lax
from jax.experimental import pallas as pl
from jax.experimental.pallas import tpu as pltpu
from jax.experimental.pallas import tpu_sc as plsc

N = 10000
D = 128
DP = D // 2
E = 320000
REG_PARAM = 0.01

NC, NS = 2, 16
NW = NC * NS
EPW = E // NW
CH = 80
NCHUNK = EPW // CH
NPAIR = (NCHUNK - 1) // 2


def _sc_scores_body(uwc_hbm, uwb_hbm, i_hbm, srcc_hbm, dstc_hbm,
                    srcb_hbm, dstb_hbm, outc_hbm, outb_hbm,
                    idxs, idxd, urowsA, irowsA, urowsB, irowsB,
                    urowsC, irowsC, urowsD, irowsD, urowsE, irowsE,
                    scores, semA, semB, semC, semD, semE):
    wid = lax.axis_index("s") * NC + lax.axis_index("c")
    base = wid * EPW
    lane = lax.iota(jnp.int32, 16)
    zbf = jnp.zeros(32, jnp.bfloat16)

    def run_etype(uw_hbm, src_hbm, dst_hbm, out_hbm):
        pltpu.sync_copy(src_hbm.at[pl.ds(base, EPW)], idxs)
        pltpu.sync_copy(dst_hbm.at[pl.ds(base, EPW)], idxd)

        def start(c, ubuf, ibuf, sem):
            pltpu.async_copy(uw_hbm.at[idxs.at[pl.ds(c * CH, CH)]], ubuf, sem)
            pltpu.async_copy(i_hbm.at[idxd.at[pl.ds(c * CH, CH)]], ibuf, sem)

        def drain(ubuf, ibuf, sem):
            pltpu.make_async_copy(uw_hbm.at[idxs.at[pl.ds(0, CH)]], ubuf, sem).wait()
            pltpu.make_async_copy(i_hbm.at[idxd.at[pl.ds(0, CH)]], ibuf, sem).wait()

        def compute(c, ubuf, ibuf):
            def group_body(g, gcarry):
                row = g * 16 + lane

                def d_body(d, carry):
                    acc, col = carry
                    up = plsc.load_gather(ubuf, [row, col])
                    ip = plsc.load_gather(ibuf, [row, col])
                    acc = acc + (plsc.bitcast(up, jnp.bfloat16)
                                 * plsc.bitcast(ip, jnp.bfloat16))
                    return (acc, (col + 1) & (DP - 1))

                acc, _ = lax.fori_loop(0, DP, d_body, (zbf, lane), unroll=8)
                hi, lo = plsc.unpack(acc, format=plsc.PackFormat.INTERLEAVED)
                scores[pl.ds(c * CH + g * 16, 16)] = hi + lo
                return gcarry

            lax.fori_loop(0, CH // 16, group_body, 0)

        bufs = [(urowsA, irowsA, semA), (urowsB, irowsB, semB),
                (urowsC, irowsC, semC), (urowsD, irowsD, semD),
                (urowsE, irowsE, semE)]
        for k in range(4):
            start(k, *bufs[k])

        def pent_body(t, carry):
            c = 5 * t
            for k in range(5):
                u, i, s = bufs[(k + 4) % 5]
                start(c + k + 4, u, i, s)
                u, i, s = bufs[k]
                drain(u, i, s)
                compute(c + k, u, i)
            return carry

        lax.fori_loop(0, 24, pent_body, 0)
        start(124, *bufs[4])
        for k in range(5):
            u, i, s = bufs[k]
            drain(u, i, s)
            compute(120 + k, u, i)
        pltpu.sync_copy(scores, out_hbm.at[pl.ds(base, EPW)])

    run_etype(uwc_hbm, srcc_hbm, dstc_hbm, outc_hbm)
    run_etype(uwb_hbm, srcb_hbm, dstb_hbm, outb_hbm)


_sc_scores = pl.kernel(
    _sc_scores_body,
    out_type=(jax.ShapeDtypeStruct((E,), jnp.float32),
              jax.ShapeDtypeStruct((E,), jnp.float32)),
    mesh=plsc.VectorSubcoreMesh(core_axis_name="c", subcore_axis_name="s",
                                num_cores=NC, num_subcores=NS),
    scratch_types=[
        pltpu.VMEM((EPW,), jnp.int32),
        pltpu.VMEM((EPW,), jnp.int32),
        pltpu.VMEM((CH, DP), jnp.int32),
        pltpu.VMEM((CH, DP), jnp.int32),
        pltpu.VMEM((CH, DP), jnp.int32),
        pltpu.VMEM((CH, DP), jnp.int32),
        pltpu.VMEM((CH, DP), jnp.int32),
        pltpu.VMEM((CH, DP), jnp.int32),
        pltpu.VMEM((CH, DP), jnp.int32),
        pltpu.VMEM((CH, DP), jnp.int32),
        pltpu.VMEM((CH, DP), jnp.int32),
        pltpu.VMEM((CH, DP), jnp.int32),
        pltpu.VMEM((EPW,), jnp.float32),
        pltpu.SemaphoreType.DMA,
        pltpu.SemaphoreType.DMA,
        pltpu.SemaphoreType.DMA,
        pltpu.SemaphoreType.DMA,
        pltpu.SemaphoreType.DMA,
    ],
    compiler_params=pltpu.CompilerParams(needs_layout_passes=False,
                                         use_tc_tiling_on_sc=False),
)


def _tc_prescale_body(u_ref, i_ref, wc_ref, wb_ref, uwc_ref, uwb_ref, ib_ref):
    u = u_ref[...]
    uwc_ref[...] = (u * wc_ref[...]).astype(jnp.bfloat16)
    uwb_ref[...] = (u * wb_ref[...]).astype(jnp.bfloat16)
    ib_ref[...] = i_ref[...].astype(jnp.bfloat16)


_tc_prescale = pl.pallas_call(
    _tc_prescale_body,
    out_shape=(jax.ShapeDtypeStruct((N, D), jnp.bfloat16),
               jax.ShapeDtypeStruct((N, D), jnp.bfloat16),
               jax.ShapeDtypeStruct((N, D), jnp.bfloat16)),
)


def _tc_loss_body(sc_ref, sb_ref, lc_ref, lb_ref, u_ref, i_ref, wc_ref, wb_ref,
                  out_ref):
    def bce_sum(s, y):
        return jnp.sum(jnp.maximum(s, 0.0) - s * y
                       + jnp.log1p(jnp.exp(-jnp.abs(s))))

    predict = (bce_sum(sc_ref[...], lc_ref[...])
               + bce_sum(sb_ref[...], lb_ref[...])) / E
    reg = (jnp.mean(u_ref[...] ** 2) + jnp.mean(i_ref[...] ** 2)
           + jnp.mean(wc_ref[...] ** 2) + jnp.mean(wb_ref[...] ** 2))
    out_ref[...] = jnp.full((1, 1), predict + REG_PARAM * reg, jnp.float32)


_tc_loss = pl.pallas_call(
    _tc_loss_body,
    out_shape=jax.ShapeDtypeStruct((1, 1), jnp.float32),
)


def _pack(x_bf16):
    return jax.lax.bitcast_convert_type(x_bf16.reshape(N, DP, 2), jnp.int32)


def kernel(embed_user, embed_item, edges_click, edges_buy, labels_click,
           labels_buy, w_click, w_buy):
    srcc = edges_click[:, 0]
    dstc = edges_click[:, 1]
    srcb = edges_buy[:, 0]
    dstb = edges_buy[:, 1]
    uwc, uwb, ib = _tc_prescale(embed_user, embed_item,
                                w_click.reshape(1, D), w_buy.reshape(1, D))
    scores_c, scores_b = _sc_scores(_pack(uwc), _pack(uwb), _pack(ib),
                                    srcc, dstc, srcb, dstb)
    out = _tc_loss(scores_c.reshape(E // D, D), scores_b.reshape(E // D, D),
                   labels_click.reshape(E // D, D), labels_buy.reshape(E // D, D),
                   embed_user, embed_item,
                   w_click.reshape(1, D), w_buy.reshape(1, D))
    return out[0, 0]


# speedup vs baseline: 1.9139x; 1.0003x over previous
import jax
import jax.numpy as jnp
from jax import lax
from jax.experimental import pallas as pl
from jax.experimental.pallas import tpu as pltpu
from jax.experimental.pallas import tpu_sc as plsc

N = 10000
D = 128
DP = D // 2
E = 320000
REG_PARAM = 0.01

NC, NS = 2, 16
NW = NC * NS
EPW = E // NW
CH = 80
NCHUNK = EPW // CH
NPAIR = (NCHUNK - 1) // 2


def _sc_scores_body(uwc_hbm, uwb_hbm, i_hbm, srcc_hbm, dstc_hbm,
                    srcb_hbm, dstb_hbm, outc_hbm, outb_hbm,
                    idxs, idxd, urowsA, irowsA, urowsB, irowsB,
                    urowsC, irowsC, urowsD, irowsD, urowsE, irowsE,
                    scores, semA, semB, semC, semD, semE):
    wid = lax.axis_index("s") * NC + lax.axis_index("c")
    base = wid * EPW
    lane = lax.iota(jnp.int32, 16)
    zbf = jnp.zeros(32, jnp.bfloat16)

    def run_etype(uw_hbm, src_hbm, dst_hbm, out_hbm):
        pltpu.sync_copy(src_hbm.at[pl.ds(base, EPW)], idxs)
        pltpu.sync_copy(dst_hbm.at[pl.ds(base, EPW)], idxd)

        def start(c, ubuf, ibuf, sem):
            pltpu.async_copy(uw_hbm.at[idxs.at[pl.ds(c * CH, CH)]], ubuf, sem)
            pltpu.async_copy(i_hbm.at[idxd.at[pl.ds(c * CH, CH)]], ibuf, sem)

        def drain(ubuf, ibuf, sem):
            pltpu.make_async_copy(uw_hbm.at[idxs.at[pl.ds(0, CH)]], ubuf, sem).wait()
            pltpu.make_async_copy(i_hbm.at[idxd.at[pl.ds(0, CH)]], ibuf, sem).wait()

        def compute(c, ubuf, ibuf):
            def group_body(g, gcarry):
                row = g * 16 + lane

                def d_body(d, carry):
                    acc, col = carry
                    up = plsc.load_gather(ubuf, [row, col])
                    ip = plsc.load_gather(ibuf, [row, col])
                    acc = acc + (plsc.bitcast(up, jnp.bfloat16)
                                 * plsc.bitcast(ip, jnp.bfloat16))
                    return (acc, (col + 1) & (DP - 1))

                acc, _ = lax.fori_loop(0, DP, d_body, (zbf, lane), unroll=8)
                hi, lo = plsc.unpack(acc, format=plsc.PackFormat.INTERLEAVED)
                scores[pl.ds(c * CH + g * 16, 16)] = hi + lo
                return gcarry

            lax.fori_loop(0, CH // 16, group_body, 0)

        bufs = [(urowsA, irowsA, semA), (urowsB, irowsB, semB),
                (urowsC, irowsC, semC), (urowsD, irowsD, semD),
                (urowsE, irowsE, semE)]
        for k in range(4):
            start(k, *bufs[k])

        def pent_body(t, carry):
            c = 5 * t
            for k in range(5):
                u, i, s = bufs[(k + 4) % 5]
                start(c + k + 4, u, i, s)
                u, i, s = bufs[k]
                drain(u, i, s)
                compute(c + k, u, i)
            return carry

        lax.fori_loop(0, 24, pent_body, 0)
        start(124, *bufs[4])
        for k in range(5):
            u, i, s = bufs[k]
            drain(u, i, s)
            compute(120 + k, u, i)
        pltpu.sync_copy(scores, out_hbm.at[pl.ds(base, EPW)])

    run_etype(uwc_hbm, srcc_hbm, dstc_hbm, outc_hbm)
    run_etype(uwb_hbm, srcb_hbm, dstb_hbm, outb_hbm)


_sc_scores = pl.kernel(
    _sc_scores_body,
    out_type=(jax.ShapeDtypeStruct((E,), jnp.float32),
              jax.ShapeDtypeStruct((E,), jnp.float32)),
    mesh=plsc.VectorSubcoreMesh(core_axis_name="c", subcore_axis_name="s",
                                num_cores=NC, num_subcores=NS),
    scratch_types=[
        pltpu.VMEM((EPW,), jnp.int32),
        pltpu.VMEM((EPW,), jnp.int32),
        pltpu.VMEM((CH, DP), jnp.int32),
        pltpu.VMEM((CH, DP), jnp.int32),
        pltpu.VMEM((CH, DP), jnp.int32),
        pltpu.VMEM((CH, DP), jnp.int32),
        pltpu.VMEM((CH, DP), jnp.int32),
        pltpu.VMEM((CH, DP), jnp.int32),
        pltpu.VMEM((CH, DP), jnp.int32),
        pltpu.VMEM((CH, DP), jnp.int32),
        pltpu.VMEM((CH, DP), jnp.int32),
        pltpu.VMEM((CH, DP), jnp.int32),
        pltpu.VMEM((EPW,), jnp.float32),
        pltpu.SemaphoreType.DMA,
        pltpu.SemaphoreType.DMA,
        pltpu.SemaphoreType.DMA,
        pltpu.SemaphoreType.DMA,
        pltpu.SemaphoreType.DMA,
    ],
    compiler_params=pltpu.CompilerParams(needs_layout_passes=False,
                                         use_tc_tiling_on_sc=False),
)


def _tc_prescale_body(u_ref, i_ref, wc_ref, wb_ref, uwc_ref, uwb_ref, ib_ref):
    u = u_ref[...]
    uwc_ref[...] = (u * wc_ref[...]).astype(jnp.bfloat16)
    uwb_ref[...] = (u * wb_ref[...]).astype(jnp.bfloat16)
    ib_ref[...] = i_ref[...].astype(jnp.bfloat16)


_tc_prescale = pl.pallas_call(
    _tc_prescale_body,
    out_shape=(jax.ShapeDtypeStruct((N, D), jnp.bfloat16),
               jax.ShapeDtypeStruct((N, D), jnp.bfloat16),
               jax.ShapeDtypeStruct((N, D), jnp.bfloat16)),
)


def _tc_loss_body(sc_ref, sb_ref, lc_ref, lb_ref, u_ref, i_ref, wc_ref, wb_ref,
                  out_ref):
    def bce_sum(s, y):
        return jnp.sum(jnp.maximum(s, 0.0) - s * y
                       + jnp.log1p(jnp.exp(-jnp.abs(s))))

    predict = (bce_sum(sc_ref[...], lc_ref[...])
               + bce_sum(sb_ref[...], lb_ref[...])) / E
    reg = (jnp.mean(u_ref[...] ** 2) + jnp.mean(i_ref[...] ** 2)
           + jnp.mean(wc_ref[...] ** 2) + jnp.mean(wb_ref[...] ** 2))
    out_ref[...] = jnp.full((1, 1), predict + REG_PARAM * reg, jnp.float32)


_tc_loss = pl.pallas_call(
    _tc_loss_body,
    out_shape=jax.ShapeDtypeStruct((1, 1), jnp.float32),
)


def _pack(x_bf16):
    return jax.lax.bitcast_convert_type(x_bf16.reshape(N, DP, 2), jnp.int32)


def kernel(embed_user, embed_item, edges_click, edges_buy, labels_click,
           labels_buy, w_click, w_buy):
    srcc = edges_click[:, 0]
    dstc = edges_click[:, 1]
    srcb = edges_buy[:, 0]
    dstb = edges_buy[:, 1]
    uwc, uwb, ib = _tc_prescale(embed_user, embed_item,
                                w_click.reshape(1, D), w_buy.reshape(1, D))
    scores_c, scores_b = _sc_scores(_pack(uwc), _pack(uwb), _pack(ib),
                                    srcc, dstc, srcb, dstb)
    out = _tc_loss(scores_c.reshape(E // D, D), scores_b.reshape(E // D, D),
                   labels_click.reshape(E // D, D), labels_buy.reshape(E // D, D),
                   embed_user, embed_item,
                   w_click.reshape(1, D), w_buy.reshape(1, D))
    return out[0, 0]
